# Initial kernel scaffold; baseline (speedup 1.0000x reference)
import jax
import jax.numpy as jnp
from jax.experimental import pallas as pl


def kernel(x, edge_index, batch, W_embed, W0, b0, gamma0, beta0, W1, b1, gamma1, beta1):
    raise NotImplementedError("write your pallas kernel here")


# speedup vs baseline: 15.1290x; 15.1290x over previous
import functools

import jax
import jax.numpy as jnp
from jax import lax
from jax.experimental import pallas as pl
from jax.experimental.pallas import tpu as pltpu
from jax.experimental.pallas import tpu_sc as plsc

N = 10000
E = 320000
IN_DIM = 128
D = 64
G = 64

NC = 2
NS = 16
NW = NC * NS
CHUNK = 128
EPT = 10112
NCHUNKS = EPT // CHUNK
E_PAD = EPT * NW
N_PAD = 10112
RPT = N_PAD // NS

_mesh = plsc.VectorSubcoreMesh(core_axis_name="c", subcore_axis_name="s")
_sc_params = pltpu.CompilerParams(use_tc_tiling_on_sc=False)


@functools.partial(
    pl.kernel,
    out_type=jax.ShapeDtypeStruct((NC * N_PAD, 16), jnp.float32),
    mesh=_mesh,
    scratch_types=[
        pltpu.VMEM((CHUNK,), jnp.int32),
        pltpu.VMEM((CHUNK, 16), jnp.float32),
        pltpu.VMEM_SHARED((N_PAD, 16), jnp.float32),
    ],
    compiler_params=_sc_params,
)
def _sc_hist(dst_hbm, ones_hbm, zeros_hbm, out_hbm, dst_v, ones_v, acc_sh):
    c = lax.axis_index("c")
    s = lax.axis_index("s")
    tid = s * NC + c
    pltpu.sync_copy(zeros_hbm.at[pl.ds(s * RPT, RPT)],
                    acc_sh.at[pl.ds(s * RPT, RPT)])
    pltpu.sync_copy(ones_hbm, ones_v)
    plsc.subcore_barrier()

    def body(i, carry):
        off = tid * EPT + i * CHUNK
        pltpu.sync_copy(dst_hbm.at[pl.ds(off, CHUNK)], dst_v)
        pltpu.sync_copy(ones_v, acc_sh.at[dst_v], add=True)
        return carry

    lax.fori_loop(0, NCHUNKS, body, 0)
    plsc.subcore_barrier()
    pltpu.sync_copy(acc_sh.at[pl.ds(s * RPT, RPT)],
                    out_hbm.at[pl.ds(c * N_PAD + s * RPT, RPT)])


@functools.partial(
    pl.kernel,
    out_type=jax.ShapeDtypeStruct((NC * N_PAD, D), jnp.float32),
    mesh=_mesh,
    scratch_types=[
        pltpu.VMEM((CHUNK,), jnp.int32),
        pltpu.VMEM((CHUNK,), jnp.int32),
        pltpu.VMEM((CHUNK, D), jnp.float32),
        pltpu.VMEM_SHARED((N_PAD, D), jnp.float32),
        pltpu.SemaphoreType.DMA,
    ],
    compiler_params=_sc_params,
)
def _sc_segsum(g_hbm, src_hbm, dst_hbm, zeros_hbm, out_hbm,
               src_v, dst_v, rows_v, acc_sh, sem):
    c = lax.axis_index("c")
    s = lax.axis_index("s")
    tid = s * NC + c
    pltpu.sync_copy(zeros_hbm.at[pl.ds(s * RPT, RPT)],
                    acc_sh.at[pl.ds(s * RPT, RPT)])
    plsc.subcore_barrier()

    def body(i, carry):
        off = tid * EPT + i * CHUNK
        pltpu.sync_copy(src_hbm.at[pl.ds(off, CHUNK)], src_v)
        pltpu.sync_copy(dst_hbm.at[pl.ds(off, CHUNK)], dst_v)
        pltpu.async_copy(g_hbm.at[src_v], rows_v, sem).wait()
        pltpu.sync_copy(rows_v, acc_sh.at[dst_v], add=True)
        return carry

    lax.fori_loop(0, NCHUNKS, body, 0)
    plsc.subcore_barrier()
    pltpu.sync_copy(acc_sh.at[pl.ds(s * RPT, RPT)],
                    out_hbm.at[pl.ds(c * N_PAD + s * RPT, RPT)])


def _tc_embed_body(x_ref, we_ref, w0_ref, d0_ref, d1_ref, g0_ref, dinv_ref):
    deg = d0_ref[...] + d1_ref[...] + 1.0
    dinv = lax.rsqrt(deg)
    dinv_ref[...] = dinv
    h = jnp.dot(x_ref[...], we_ref[...], preferred_element_type=jnp.float32)
    t = jnp.dot(h, w0_ref[...], preferred_element_type=jnp.float32)
    g0_ref[...] = dinv * t


def _tc_mid_body(p0_ref, p1_ref, g_ref, dinv_ref, b_ref, gam_ref, bet_ref,
                 w1_ref, g1_ref):
    dinv = dinv_ref[...]
    u = dinv * (p0_ref[...] + p1_ref[...] + g_ref[...]) + b_ref[...]
    mu = jnp.mean(u, axis=0, keepdims=True)
    var = jnp.mean((u - mu) ** 2, axis=0, keepdims=True)
    h = (u - mu) * lax.rsqrt(var + 1e-5) * gam_ref[...] + bet_ref[...]
    h = jnp.maximum(h, 0.0)
    t = jnp.dot(h, w1_ref[...], preferred_element_type=jnp.float32)
    g1_ref[...] = dinv * t


def _tc_final_body(p0_ref, p1_ref, g_ref, dinv_ref, b_ref, gam_ref, bet_ref,
                   batch_ref, h_ref, pool_ref):
    dinv = dinv_ref[...]
    u = dinv * (p0_ref[...] + p1_ref[...] + g_ref[...]) + b_ref[...]
    mu = jnp.mean(u, axis=0, keepdims=True)
    var = jnp.mean((u - mu) ** 2, axis=0, keepdims=True)
    h = (u - mu) * lax.rsqrt(var + 1e-5) * gam_ref[...] + bet_ref[...]
    h_ref[...] = h
    gids = lax.broadcasted_iota(jnp.int32, (G, N), 0)
    onehot = (gids == batch_ref[...]).astype(jnp.float32)
    pool_ref[...] = jnp.dot(onehot, h, preferred_element_type=jnp.float32)


def kernel(x, edge_index, batch, W_embed, W0, b0, gamma0, beta0,
           W1, b1, gamma1, beta1):
    src = edge_index[0]
    dst = edge_index[1]
    pad = E_PAD - E
    src_p = jnp.concatenate([src, jnp.zeros((pad,), jnp.int32)])
    dst_p = jnp.concatenate([dst, jnp.full((pad,), N, jnp.int32)])
    ones16 = jnp.ones((CHUNK, 16), jnp.float32)
    zeros16 = jnp.zeros((N_PAD, 16), jnp.float32)
    zerosD = jnp.zeros((N_PAD, D), jnp.float32)

    degp = _sc_hist(dst_p, ones16, zeros16)
    d0 = degp[0 * N_PAD:0 * N_PAD + N, 0:1]
    d1 = degp[1 * N_PAD:1 * N_PAD + N, 0:1]

    g0, dinv = pl.pallas_call(
        _tc_embed_body,
        out_shape=(jax.ShapeDtypeStruct((N, D), jnp.float32),
                   jax.ShapeDtypeStruct((N, 1), jnp.float32)),
    )(x, W_embed, W0, d0, d1)

    s0 = _sc_segsum(g0, src_p, dst_p, zerosD)

    g1 = pl.pallas_call(
        _tc_mid_body,
        out_shape=jax.ShapeDtypeStruct((N, D), jnp.float32),
    )(s0[:N], s0[N_PAD:N_PAD + N], g0, dinv,
      b0.reshape(1, D), gamma0.reshape(1, D), beta0.reshape(1, D), W1)

    s1 = _sc_segsum(g1, src_p, dst_p, zerosD)

    h, pool = pl.pallas_call(
        _tc_final_body,
        out_shape=(jax.ShapeDtypeStruct((N, D), jnp.float32),
                   jax.ShapeDtypeStruct((G, D), jnp.float32)),
    )(s1[:N], s1[N_PAD:N_PAD + N], g1, dinv,
      b1.reshape(1, D), gamma1.reshape(1, D), beta1.reshape(1, D),
      batch.reshape(1, N))

    return (h, pool)
